# baseline (speedup 1.0000x reference)
import jax
import jax.numpy as jnp
from jax.experimental import pallas as pl
from jax.experimental.pallas import tpu as pltpu
from jax.experimental.pallas import tpu_sc as plsc

_EDGE_IN = 19
_N_OUT = 2
_SC_TILES = 32
_SLICES = 2
_TILE_LANES = 2560
_CHUNKS = _TILE_LANES // 16


def _sc_body(wb_hbm, et_hbm, o_hbm, wbuf, ebuf, obuf, sems):
    c = jax.lax.axis_index("c")
    s = jax.lax.axis_index("s")
    t = c * 16 + s
    max_off = et_hbm.shape[1] - _TILE_LANES

    cp_w = pltpu.make_async_copy(wb_hbm, wbuf, sems.at[0])
    cp_w.start()
    cp_w.wait()

    r0a = wbuf[0, pl.ds(0, 16)]
    r0b = wbuf[0, pl.ds(16, 16)]
    r1a = wbuf[1, pl.ds(0, 16)]
    r1b = wbuf[1, pl.ds(16, 16)]
    w0 = [r0a[k] if k < 16 else r0b[k - 16] for k in range(_EDGE_IN)]
    w1 = [r1a[k] if k < 16 else r1b[k - 16] for k in range(_EDGE_IN)]
    b0 = r0b[_EDGE_IN - 16]
    b1 = r1b[_EDGE_IN - 16]

    def chunk(i, carry):
        sl = pl.ds(i * 16, 16)
        acc0 = jnp.full((16,), b0, jnp.float32)
        acc1 = jnp.full((16,), b1, jnp.float32)
        for k in range(_EDGE_IN):
            v = ebuf[k, sl]
            acc0 = acc0 + v * w0[k]
            acc1 = acc1 + v * w1[k]
        obuf[0, sl] = acc0
        obuf[1, sl] = acc1
        return carry

    for r in range(_SLICES):
        j = t * _SLICES + r
        off = jnp.minimum(j * _TILE_LANES, max_off)
        cp_e = pltpu.make_async_copy(
            et_hbm.at[:, pl.ds(off, _TILE_LANES)], ebuf, sems.at[1]
        )
        cp_e.start()
        cp_e.wait()
        jax.lax.fori_loop(0, _CHUNKS, chunk, 0)
        cp_o = pltpu.make_async_copy(
            obuf, o_hbm.at[:, pl.ds(off, _TILE_LANES)], sems.at[2]
        )
        cp_o.start()
        cp_o.wait()


def kernel(x, edge_index, e, xbatch, bn_g0, bn_b0, W00, b00, W01, b01,
           root0, rb0, bn_g1, bn_b1, W10, b10, W11, b11, root1, rb1,
           bn_g2, bn_b2, W20, b20, W21, b21, root2, rb2, Wp, bp):
    e = e.reshape(-1, _EDGE_IN)
    n_edges = e.shape[0]
    n_out = Wp.shape[1]
    et = e.T
    wb = jnp.concatenate(
        [Wp.T, bp.reshape(n_out, 1), jnp.zeros((n_out, 12), jnp.float32)],
        axis=1,
    )

    sck = pl.kernel(
        _sc_body,
        out_type=jax.ShapeDtypeStruct((n_out, n_edges), jnp.float32),
        mesh=plsc.VectorSubcoreMesh(core_axis_name="c", subcore_axis_name="s"),
        scratch_types=[
            pltpu.VMEM((_N_OUT, 32), jnp.float32),
            pltpu.VMEM((_EDGE_IN, _TILE_LANES), jnp.float32),
            pltpu.VMEM((_N_OUT, _TILE_LANES), jnp.float32),
            pltpu.SemaphoreType.DMA((3,)),
        ],
        compiler_params=pltpu.CompilerParams(use_tc_tiling_on_sc=True),
    )
    out_t = sck(wb, et)
    return out_t.T


# speedup vs baseline: 3.7962x; 3.7962x over previous
import jax
import jax.numpy as jnp
from jax.experimental import pallas as pl
from jax.experimental.pallas import tpu as pltpu

_EDGE_IN = 19
_SLABS = ((0, 8), (8, 8), (16, 3))


def _edge_pred_kernel(w_ref, b_ref, et_hbm, o_ref, ebuf, sems):
    copies = [
        pltpu.make_async_copy(
            et_hbm.at[pl.ds(lo, n), :], ebuf.at[pl.ds(lo, n), :], sems.at[i]
        )
        for i, (lo, n) in enumerate(_SLABS)
    ]
    for c in copies:
        c.start()
    for c in copies:
        c.wait()
    o_ref[...] = (
        jnp.dot(w_ref[...], ebuf[...], preferred_element_type=jnp.float32)
        + b_ref[...]
    )


def kernel(x, edge_index, e, xbatch, bn_g0, bn_b0, W00, b00, W01, b01,
           root0, rb0, bn_g1, bn_b1, W10, b10, W11, b11, root1, rb1,
           bn_g2, bn_b2, W20, b20, W21, b21, root2, rb2, Wp, bp):
    e = e.reshape(-1, _EDGE_IN)
    n_edges = e.shape[0]
    n_out = Wp.shape[1]
    et = e.T
    wt = Wp.T
    bias = bp.reshape(n_out, 1)

    out_t = pl.pallas_call(
        _edge_pred_kernel,
        in_specs=[
            pl.BlockSpec(memory_space=pltpu.VMEM),
            pl.BlockSpec(memory_space=pltpu.VMEM),
            pl.BlockSpec(memory_space=pl.ANY),
        ],
        out_specs=pl.BlockSpec(memory_space=pltpu.VMEM),
        out_shape=jax.ShapeDtypeStruct((n_out, n_edges), jnp.float32),
        scratch_shapes=[
            pltpu.VMEM((_EDGE_IN, n_edges), jnp.float32),
            pltpu.SemaphoreType.DMA((len(_SLABS),)),
        ],
    )(wt, bias, et)
    return out_t.T


# speedup vs baseline: 4.2035x; 1.1073x over previous
import jax
import jax.numpy as jnp
from jax.experimental import pallas as pl
from jax.experimental.pallas import tpu as pltpu

_EDGE_IN = 19
_CHUNKS = 10


def _edge_pred_kernel(w_ref, b_ref, et_ref, o_hbm, obuf, sems):
    n_edges = et_ref.shape[1]
    chunk = n_edges // _CHUNKS
    w = w_ref[...]
    b = b_ref[...]
    for i in range(_CHUNKS):
        sl = pl.ds(i * chunk, chunk)
        obuf[:, sl] = (
            jnp.dot(w, et_ref[:, sl], preferred_element_type=jnp.float32) + b
        )
        pltpu.make_async_copy(
            obuf.at[:, sl], o_hbm.at[:, sl], sems.at[i]
        ).start()
    for i in range(_CHUNKS):
        pltpu.make_async_copy(
            obuf.at[:, pl.ds(i * chunk, chunk)],
            o_hbm.at[:, pl.ds(i * chunk, chunk)],
            sems.at[i],
        ).wait()


def kernel(x, edge_index, e, xbatch, bn_g0, bn_b0, W00, b00, W01, b01,
           root0, rb0, bn_g1, bn_b1, W10, b10, W11, b11, root1, rb1,
           bn_g2, bn_b2, W20, b20, W21, b21, root2, rb2, Wp, bp):
    e = e.reshape(-1, _EDGE_IN)
    n_edges = e.shape[0]
    n_out = Wp.shape[1]
    et = e.T
    wt = Wp.T
    bias = bp.reshape(n_out, 1)

    out_t = pl.pallas_call(
        _edge_pred_kernel,
        in_specs=[
            pl.BlockSpec(memory_space=pltpu.VMEM),
            pl.BlockSpec(memory_space=pltpu.VMEM),
            pl.BlockSpec(memory_space=pltpu.VMEM),
        ],
        out_specs=pl.BlockSpec(memory_space=pl.ANY),
        out_shape=jax.ShapeDtypeStruct((n_out, n_edges), jnp.float32),
        scratch_shapes=[
            pltpu.VMEM((n_out, n_edges), jnp.float32),
            pltpu.SemaphoreType.DMA((_CHUNKS,)),
        ],
    )(wt, bias, et)
    return out_t.T
